# Initial kernel scaffold; baseline (speedup 1.0000x reference)
import jax
import jax.numpy as jnp
from jax.experimental import pallas as pl


def kernel(adjacency_matrix, feats, W1_w, W1_b, a1_w, a1_b, W2_w, W2_b, a2_w, a2_b, out_w, out_b):
    raise NotImplementedError("write your pallas kernel here")


# speedup vs baseline: 2.5974x; 2.5974x over previous
import jax
import jax.numpy as jnp
from jax.experimental import pallas as pl
from jax.experimental.pallas import tpu as pltpu


def _h1_kernel(adj_ref, feats_ref, w1_ref, b1_ref, out_ref):
    fw = jnp.dot(feats_ref[...], w1_ref[...], preferred_element_type=jnp.float32)
    out_ref[...] = (
        jnp.dot(adj_ref[...], fw, preferred_element_type=jnp.float32) + b1_ref[...]
    )


def _attn_heads(adj, h, hblk, ht, asrc, adst, ab):
    deg = jnp.sum(adj, axis=1, keepdims=True)
    has = deg > 0.0
    src_all = jnp.dot(hblk, asrc, preferred_element_type=jnp.float32) + ab
    dst_all = jnp.dot(adst, ht, preferred_element_type=jnp.float32)
    heads = src_all.shape[1]
    outs = []
    for k in range(heads):
        z = src_all[:, k : k + 1] + dst_all[k : k + 1, :]
        e = jnp.maximum(z, 0.01 * z)
        c = jnp.max(e, axis=1, keepdims=True)
        p = jnp.exp(e - c) * adj
        denom = jnp.sum(p, axis=1, keepdims=True)
        agg = jnp.dot(p, h, preferred_element_type=jnp.float32) / jnp.where(
            denom > 0.0, denom, 1.0
        )
        outs.append(hblk + jnp.where(has, agg, 0.0))
    return outs


def _elu(x):
    return jnp.where(x > 0.0, x, jnp.exp(x) - 1.0)


def _attn1_kernel(adj_ref, h_ref, hblk_ref, ht_ref, asrc_ref, adst_ref, ab_ref,
                  w2_ref, b2_ref, out_ref):
    outs = _attn_heads(adj_ref[...], h_ref[...], hblk_ref[...], ht_ref[...],
                       asrc_ref[...], adst_ref[...], ab_ref[...])
    cat = _elu(jnp.concatenate(outs, axis=1))
    out_ref[...] = (
        jnp.dot(cat, w2_ref[...], preferred_element_type=jnp.float32) + b2_ref[...]
    )


def _attn2_kernel(adj_ref, h_ref, hblk_ref, ht_ref, asrc_ref, adst_ref, ab_ref,
                  ow_ref, ob_ref, inv_n_ref, out_ref, acc_ref):
    i = pl.program_id(0)
    outs = _attn_heads(adj_ref[...], h_ref[...], hblk_ref[...], ht_ref[...],
                       asrc_ref[...], adst_ref[...], ab_ref[...])
    avg = (outs[0] + outs[1] + outs[2] + outs[3]) * 0.25
    part = jnp.sum(_elu(avg), axis=0, keepdims=True)

    @pl.when(i == 0)
    def _():
        acc_ref[...] = jnp.zeros_like(acc_ref)

    acc_ref[...] += part

    @pl.when(i == pl.num_programs(0) - 1)
    def _():
        avgd = acc_ref[...] * inv_n_ref[...]
        out_ref[...] = (
            jnp.dot(avgd, ow_ref[...], preferred_element_type=jnp.float32)
            + ob_ref[...]
        )


def kernel(adjacency_matrix, feats, W1_w, W1_b, a1_w, a1_b, W2_w, W2_b,
           a2_w, a2_b, out_w, out_b):
    n = adjacency_matrix.shape[0]
    d_feat = feats.shape[1]
    hid = W1_w.shape[1]
    heads = a1_w.shape[0]
    br = 256
    nb = n // br

    full = lambda r, c: pl.BlockSpec((r, c), lambda i: (0, 0))
    rows = lambda c: pl.BlockSpec((br, c), lambda i: (i, 0))

    h1 = pl.pallas_call(
        _h1_kernel,
        grid=(nb,),
        in_specs=[rows(n), full(n, d_feat), full(d_feat, hid), full(1, hid)],
        out_specs=rows(hid),
        out_shape=jax.ShapeDtypeStruct((n, hid), jnp.float32),
    )(adjacency_matrix, feats, W1_w, W1_b.reshape(1, hid))

    def attn_layer1(h, aw, ab_, w2, b2):
        return pl.pallas_call(
            _attn1_kernel,
            grid=(nb,),
            in_specs=[rows(n), full(n, hid), rows(hid), full(hid, n),
                      full(hid, heads), full(heads, hid), full(1, heads),
                      full(heads * hid, hid), full(1, hid)],
            out_specs=rows(hid),
            out_shape=jax.ShapeDtypeStruct((n, hid), jnp.float32),
        )(adjacency_matrix, h, h, h.T, aw[:, :hid].T, aw[:, hid:],
          ab_.reshape(1, heads), w2, b2.reshape(1, hid))

    h2 = attn_layer1(h1, a1_w, a1_b, W2_w, W2_b)

    res = pl.pallas_call(
        _attn2_kernel,
        grid=(nb,),
        in_specs=[rows(n), full(n, hid), rows(hid), full(hid, n),
                  full(hid, heads), full(heads, hid), full(1, heads),
                  full(hid, 1), full(1, 1), full(1, 1)],
        out_specs=pl.BlockSpec((1, 1), lambda i: (0, 0)),
        out_shape=jax.ShapeDtypeStruct((1, 1), jnp.float32),
        scratch_shapes=[pltpu.VMEM((1, hid), jnp.float32)],
    )(adjacency_matrix, h2, h2, h2.T, a2_w[:, :hid].T, a2_w[:, hid:],
      a2_b.reshape(1, heads), out_w, out_b.reshape(1, 1),
      jnp.full((1, 1), 1.0 / n, dtype=jnp.float32))

    return res.reshape(1)


# speedup vs baseline: 3.7420x; 1.4407x over previous
import jax
import jax.numpy as jnp
from jax.experimental import pallas as pl
from jax.experimental.pallas import tpu as pltpu


def _h1_kernel(adj_ref, feats_ref, w1_ref, b1_ref, out_ref):
    fw = jnp.dot(feats_ref[...], w1_ref[...], preferred_element_type=jnp.float32)
    out_ref[...] = (
        jnp.dot(adj_ref[...], fw, preferred_element_type=jnp.float32) + b1_ref[...]
    )


def _attn_heads(adj, haug, hblk, ht, asrc, adst, ab):
    hid = hblk.shape[1]
    deg = jnp.sum(adj, axis=1, keepdims=True)
    has = deg > 0.0
    src_all = jnp.dot(hblk, asrc, preferred_element_type=jnp.float32) + ab
    dst_all = jnp.dot(adst, ht, preferred_element_type=jnp.float32)
    heads = src_all.shape[1]
    outs = []
    for k in range(heads):
        dst = dst_all[k : k + 1, :]
        dmax = jnp.max(dst)
        src = src_all[:, k : k + 1]
        c = jnp.maximum(src + dmax, 0.0)
        u = (src - c) + dst
        p = jnp.exp(jnp.maximum(u, 0.01 * u - 0.99 * c)) * adj
        agg_den = jnp.dot(p, haug, preferred_element_type=jnp.float32)
        denom = agg_den[:, hid : hid + 1]
        agg = agg_den[:, :hid] / jnp.where(denom > 0.0, denom, 1.0)
        outs.append(hblk + jnp.where(has, agg, 0.0))
    return outs


def _elu(x):
    return jnp.where(x > 0.0, x, jnp.exp(x) - 1.0)


def _attn1_kernel(adj_ref, haug_ref, hblk_ref, ht_ref, asrc_ref, adst_ref, ab_ref,
                  w2_ref, b2_ref, out_ref):
    outs = _attn_heads(adj_ref[...], haug_ref[...], hblk_ref[...], ht_ref[...],
                       asrc_ref[...], adst_ref[...], ab_ref[...])
    cat = _elu(jnp.concatenate(outs, axis=1))
    out_ref[...] = (
        jnp.dot(cat, w2_ref[...], preferred_element_type=jnp.float32) + b2_ref[...]
    )


def _attn2_kernel(adj_ref, haug_ref, hblk_ref, ht_ref, asrc_ref, adst_ref, ab_ref,
                  ow_ref, ob_ref, inv_n_ref, out_ref, acc_ref):
    i = pl.program_id(0)
    outs = _attn_heads(adj_ref[...], haug_ref[...], hblk_ref[...], ht_ref[...],
                       asrc_ref[...], adst_ref[...], ab_ref[...])
    avg = (outs[0] + outs[1] + outs[2] + outs[3]) * 0.25
    part = jnp.sum(_elu(avg), axis=0, keepdims=True)

    @pl.when(i == 0)
    def _():
        acc_ref[...] = jnp.zeros_like(acc_ref)

    acc_ref[...] += part

    @pl.when(i == pl.num_programs(0) - 1)
    def _():
        avgd = acc_ref[...] * inv_n_ref[...]
        out_ref[...] = (
            jnp.dot(avgd, ow_ref[...], preferred_element_type=jnp.float32)
            + ob_ref[...]
        )


def kernel(adjacency_matrix, feats, W1_w, W1_b, a1_w, a1_b, W2_w, W2_b,
           a2_w, a2_b, out_w, out_b):
    n = adjacency_matrix.shape[0]
    d_feat = feats.shape[1]
    hid = W1_w.shape[1]
    heads = a1_w.shape[0]
    br = 256
    nb = n // br

    full = lambda r, c: pl.BlockSpec((r, c), lambda i: (0, 0))
    rows = lambda c: pl.BlockSpec((br, c), lambda i: (i, 0))

    h1 = pl.pallas_call(
        _h1_kernel,
        grid=(nb,),
        in_specs=[rows(n), full(n, d_feat), full(d_feat, hid), full(1, hid)],
        out_specs=rows(hid),
        out_shape=jax.ShapeDtypeStruct((n, hid), jnp.float32),
    )(adjacency_matrix, feats, W1_w, W1_b.reshape(1, hid))

    ones_col = jnp.ones((n, 1), dtype=jnp.float32)

    h2 = pl.pallas_call(
        _attn1_kernel,
        grid=(nb,),
        in_specs=[rows(n), full(n, hid + 1), rows(hid), full(hid, n),
                  full(hid, heads), full(heads, hid), full(1, heads),
                  full(heads * hid, hid), full(1, hid)],
        out_specs=rows(hid),
        out_shape=jax.ShapeDtypeStruct((n, hid), jnp.float32),
    )(adjacency_matrix, jnp.concatenate([h1, ones_col], axis=1), h1, h1.T,
      a1_w[:, :hid].T, a1_w[:, hid:], a1_b.reshape(1, heads),
      W2_w, W2_b.reshape(1, hid))

    res = pl.pallas_call(
        _attn2_kernel,
        grid=(nb,),
        in_specs=[rows(n), full(n, hid + 1), rows(hid), full(hid, n),
                  full(hid, heads), full(heads, hid), full(1, heads),
                  full(hid, 1), full(1, 1), full(1, 1)],
        out_specs=pl.BlockSpec((1, 1), lambda i: (0, 0)),
        out_shape=jax.ShapeDtypeStruct((1, 1), jnp.float32),
        scratch_shapes=[pltpu.VMEM((1, hid), jnp.float32)],
    )(adjacency_matrix, jnp.concatenate([h2, ones_col], axis=1), h2, h2.T,
      a2_w[:, :hid].T, a2_w[:, hid:], a2_b.reshape(1, heads),
      out_w, out_b.reshape(1, 1),
      jnp.full((1, 1), 1.0 / n, dtype=jnp.float32))

    return res.reshape(1)


# speedup vs baseline: 4.3658x; 1.1667x over previous
import jax
import jax.numpy as jnp
from jax.experimental import pallas as pl
from jax.experimental.pallas import tpu as pltpu


def _h1_kernel(adj_ref, feats_ref, w1_ref, b1_ref, out_ref):
    fw = jnp.dot(feats_ref[...], w1_ref[...], preferred_element_type=jnp.float32)
    out_ref[...] = (
        jnp.dot(adj_ref[...], fw, preferred_element_type=jnp.float32) + b1_ref[...]
    )


def _attn_heads(adj, haug, hblk, ht, asrc, adst, ab):
    hid = hblk.shape[1]
    deg = jnp.sum(adj, axis=1, keepdims=True)
    has = deg > 0.0
    src_all = jnp.dot(hblk, asrc, preferred_element_type=jnp.float32) + ab
    dst_all = jnp.dot(adst, ht, preferred_element_type=jnp.float32)
    heads = src_all.shape[1]
    lam = 1.4426950408889634
    outs = []
    for k in range(heads):
        dst = dst_all[k : k + 1, :]
        dmax = jnp.max(dst)
        src = src_all[:, k : k + 1]
        c = jnp.maximum(src + dmax, 0.0)
        s1 = lam * (src - c)
        d1 = lam * dst
        s2 = (0.01 * lam) * src - lam * c
        d2 = (0.01 * lam) * dst
        p = jnp.exp2(jnp.maximum(s1 + d1, s2 + d2)) * adj
        agg_den = jnp.dot(p, haug, preferred_element_type=jnp.float32)
        denom = agg_den[:, hid : hid + 1]
        agg = agg_den[:, :hid] / jnp.where(denom > 0.0, denom, 1.0)
        outs.append(hblk + jnp.where(has, agg, 0.0))
    return outs


def _elu(x):
    return jnp.where(x > 0.0, x, jnp.exp(x) - 1.0)


def _attn1_kernel(adj_ref, haug_ref, hblk_ref, ht_ref, asrc_ref, adst_ref, ab_ref,
                  w2_ref, b2_ref, out_ref):
    outs = _attn_heads(adj_ref[...], haug_ref[...], hblk_ref[...], ht_ref[...],
                       asrc_ref[...], adst_ref[...], ab_ref[...])
    cat = _elu(jnp.concatenate(outs, axis=1))
    out_ref[...] = (
        jnp.dot(cat, w2_ref[...], preferred_element_type=jnp.float32) + b2_ref[...]
    )


def _attn2_kernel(adj_ref, haug_ref, hblk_ref, ht_ref, asrc_ref, adst_ref, ab_ref,
                  ow_ref, ob_ref, inv_n_ref, out_ref, acc_ref):
    i = pl.program_id(0)
    outs = _attn_heads(adj_ref[...], haug_ref[...], hblk_ref[...], ht_ref[...],
                       asrc_ref[...], adst_ref[...], ab_ref[...])
    avg = (outs[0] + outs[1] + outs[2] + outs[3]) * 0.25
    part = jnp.sum(_elu(avg), axis=0, keepdims=True)

    @pl.when(i == 0)
    def _():
        acc_ref[...] = jnp.zeros_like(acc_ref)

    acc_ref[...] += part

    @pl.when(i == pl.num_programs(0) - 1)
    def _():
        avgd = acc_ref[...] * inv_n_ref[...]
        out_ref[...] = (
            jnp.dot(avgd, ow_ref[...], preferred_element_type=jnp.float32)
            + ob_ref[...]
        )


def kernel(adjacency_matrix, feats, W1_w, W1_b, a1_w, a1_b, W2_w, W2_b,
           a2_w, a2_b, out_w, out_b):
    n = adjacency_matrix.shape[0]
    d_feat = feats.shape[1]
    hid = W1_w.shape[1]
    heads = a1_w.shape[0]
    br = 256
    nb = n // br

    full = lambda r, c: pl.BlockSpec((r, c), lambda i: (0, 0))
    rows = lambda c: pl.BlockSpec((br, c), lambda i: (i, 0))

    h1 = pl.pallas_call(
        _h1_kernel,
        grid=(nb,),
        in_specs=[rows(n), full(n, d_feat), full(d_feat, hid), full(1, hid)],
        out_specs=rows(hid),
        out_shape=jax.ShapeDtypeStruct((n, hid), jnp.float32),
    )(adjacency_matrix, feats, W1_w, W1_b.reshape(1, hid))

    ones_col = jnp.ones((n, 1), dtype=jnp.float32)

    h2 = pl.pallas_call(
        _attn1_kernel,
        grid=(nb,),
        in_specs=[rows(n), full(n, hid + 1), rows(hid), full(hid, n),
                  full(hid, heads), full(heads, hid), full(1, heads),
                  full(heads * hid, hid), full(1, hid)],
        out_specs=rows(hid),
        out_shape=jax.ShapeDtypeStruct((n, hid), jnp.float32),
    )(adjacency_matrix, jnp.concatenate([h1, ones_col], axis=1), h1, h1.T,
      a1_w[:, :hid].T, a1_w[:, hid:], a1_b.reshape(1, heads),
      W2_w, W2_b.reshape(1, hid))

    res = pl.pallas_call(
        _attn2_kernel,
        grid=(nb,),
        in_specs=[rows(n), full(n, hid + 1), rows(hid), full(hid, n),
                  full(hid, heads), full(heads, hid), full(1, heads),
                  full(hid, 1), full(1, 1), full(1, 1)],
        out_specs=pl.BlockSpec((1, 1), lambda i: (0, 0)),
        out_shape=jax.ShapeDtypeStruct((1, 1), jnp.float32),
        scratch_shapes=[pltpu.VMEM((1, hid), jnp.float32)],
    )(adjacency_matrix, jnp.concatenate([h2, ones_col], axis=1), h2, h2.T,
      a2_w[:, :hid].T, a2_w[:, hid:], a2_b.reshape(1, heads),
      out_w, out_b.reshape(1, 1),
      jnp.full((1, 1), 1.0 / n, dtype=jnp.float32))

    return res.reshape(1)


# speedup vs baseline: 4.4300x; 1.0147x over previous
import jax
import jax.numpy as jnp
from jax.experimental import pallas as pl
from jax.experimental.pallas import tpu as pltpu


def _h1_kernel(adj_ref, feats_ref, w1_ref, b1_ref, out_ref):
    am = jnp.dot(adj_ref[...], feats_ref[...], preferred_element_type=jnp.float32)
    out_ref[...] = (
        jnp.dot(am, w1_ref[...], preferred_element_type=jnp.float32) + b1_ref[...]
    )


def _attn_heads(adj, haug, hblk, ht, asrc, adst, ab):
    hid = hblk.shape[1]
    deg = jnp.sum(adj, axis=1, keepdims=True)
    has = deg > 0.0
    src_all = jnp.dot(hblk, asrc, preferred_element_type=jnp.float32) + ab
    dst_all = jnp.dot(adst, ht, preferred_element_type=jnp.float32)
    heads = src_all.shape[1]
    lam = 1.4426950408889634
    outs = []
    for k in range(heads):
        dst = dst_all[k : k + 1, :]
        dmax = jnp.max(dst)
        src = src_all[:, k : k + 1]
        c = jnp.maximum(src + dmax, 0.0)
        s1 = lam * (src - c)
        d1 = lam * dst
        s2 = (0.01 * lam) * src - lam * c
        d2 = (0.01 * lam) * dst
        p = jnp.exp2(jnp.maximum(s1 + d1, s2 + d2)) * adj
        agg_den = jnp.dot(p, haug, preferred_element_type=jnp.float32)
        denom = agg_den[:, hid : hid + 1]
        agg = agg_den[:, :hid] / jnp.where(denom > 0.0, denom, 1.0)
        outs.append(hblk + jnp.where(has, agg, 0.0))
    return outs


def _elu(x):
    return jnp.where(x > 0.0, x, jnp.exp(x) - 1.0)


def _attn1_kernel(adj_ref, haug_ref, hblk_ref, ht_ref, asrc_ref, adst_ref, ab_ref,
                  w2_ref, b2_ref, out_ref):
    outs = _attn_heads(adj_ref[...], haug_ref[...], hblk_ref[...], ht_ref[...],
                       asrc_ref[...], adst_ref[...], ab_ref[...])
    cat = _elu(jnp.concatenate(outs, axis=1))
    out_ref[...] = (
        jnp.dot(cat, w2_ref[...], preferred_element_type=jnp.float32) + b2_ref[...]
    )


def _attn2_kernel(adj_ref, haug_ref, hblk_ref, ht_ref, asrc_ref, adst_ref, ab_ref,
                  ow_ref, ob_ref, inv_n_ref, out_ref, acc_ref):
    i = pl.program_id(0)
    outs = _attn_heads(adj_ref[...], haug_ref[...], hblk_ref[...], ht_ref[...],
                       asrc_ref[...], adst_ref[...], ab_ref[...])
    avg = (outs[0] + outs[1] + outs[2] + outs[3]) * 0.25
    part = jnp.sum(_elu(avg), axis=0, keepdims=True)

    @pl.when(i == 0)
    def _():
        acc_ref[...] = jnp.zeros_like(acc_ref)

    acc_ref[...] += part

    @pl.when(i == pl.num_programs(0) - 1)
    def _():
        avgd = acc_ref[...] * inv_n_ref[...]
        out_ref[...] = (
            jnp.dot(avgd, ow_ref[...], preferred_element_type=jnp.float32)
            + ob_ref[...]
        )


def kernel(adjacency_matrix, feats, W1_w, W1_b, a1_w, a1_b, W2_w, W2_b,
           a2_w, a2_b, out_w, out_b):
    n = adjacency_matrix.shape[0]
    d_feat = feats.shape[1]
    hid = W1_w.shape[1]
    heads = a1_w.shape[0]
    br = 256
    nb = n // br

    full = lambda r, c: pl.BlockSpec((r, c), lambda i: (0, 0))
    rows = lambda c: pl.BlockSpec((br, c), lambda i: (i, 0))

    h1 = pl.pallas_call(
        _h1_kernel,
        grid=(nb,),
        in_specs=[rows(n), full(n, d_feat), full(d_feat, hid), full(1, hid)],
        out_specs=rows(hid),
        out_shape=jax.ShapeDtypeStruct((n, hid), jnp.float32),
    )(adjacency_matrix, feats, W1_w, W1_b.reshape(1, hid))

    ones_col = jnp.ones((n, 1), dtype=jnp.float32)

    h2 = pl.pallas_call(
        _attn1_kernel,
        grid=(nb,),
        in_specs=[rows(n), full(n, hid + 1), rows(hid), full(hid, n),
                  full(hid, heads), full(heads, hid), full(1, heads),
                  full(heads * hid, hid), full(1, hid)],
        out_specs=rows(hid),
        out_shape=jax.ShapeDtypeStruct((n, hid), jnp.float32),
    )(adjacency_matrix, jnp.concatenate([h1, ones_col], axis=1), h1, h1.T,
      a1_w[:, :hid].T, a1_w[:, hid:], a1_b.reshape(1, heads),
      W2_w, W2_b.reshape(1, hid))

    res = pl.pallas_call(
        _attn2_kernel,
        grid=(nb,),
        in_specs=[rows(n), full(n, hid + 1), rows(hid), full(hid, n),
                  full(hid, heads), full(heads, hid), full(1, heads),
                  full(hid, 1), full(1, 1), full(1, 1)],
        out_specs=pl.BlockSpec((1, 1), lambda i: (0, 0)),
        out_shape=jax.ShapeDtypeStruct((1, 1), jnp.float32),
        scratch_shapes=[pltpu.VMEM((1, hid), jnp.float32)],
    )(adjacency_matrix, jnp.concatenate([h2, ones_col], axis=1), h2, h2.T,
      a2_w[:, :hid].T, a2_w[:, hid:], a2_b.reshape(1, heads),
      out_w, out_b.reshape(1, 1),
      jnp.full((1, 1), 1.0 / n, dtype=jnp.float32))

    return res.reshape(1)
